# baseline (speedup 1.0000x reference)
import functools

import jax
import jax.numpy as jnp
from jax import lax
from jax.experimental import pallas as pl
from jax.experimental.pallas import tpu as pltpu
from jax.experimental.pallas import tpu_sc as plsc

N, D, M, K = 4096, 32, 512, 16
JITTER = 1e-4
RB = 256
NW = 32
RW = N // NW
NP = K * K
TROWS = 2 * M + 1
GROWS = 2 * NP + K
SUB = N // 512
DEPTH = 16


def _tables_body(z_ref, lu_raw_ref, th_ref, td_ref, lu_ref):
    z = z_ref[...]
    zn = jnp.sum(z * z, axis=1)
    g = jnp.dot(z, z.T, preferred_element_type=jnp.float32)
    d2 = jnp.maximum(zn[:, None] + zn[None, :] - 2.0 * g, 0.0)
    kzz = jnp.exp(-0.5 * d2)
    ri = lax.broadcasted_iota(jnp.int32, (M, M), 0)
    ci = lax.broadcasted_iota(jnp.int32, (M, M), 1)
    eye = (ri == ci).astype(jnp.float32)
    lur = lu_raw_ref[...]
    lu = jnp.where(ri > ci, lur, 0.0) + eye * jnp.exp(lur * eye)
    s = jnp.dot(lu, lu.T, preferred_element_type=jnp.float32)
    th_ref[...] = kzz + (2.0 * JITTER) * eye
    td_ref[...] = s - kzz - JITTER * eye
    lu_ref[...] = lu


_tables = pl.pallas_call(
    _tables_body,
    out_shape=(
        jax.ShapeDtypeStruct((M, M), jnp.float32),
        jax.ShapeDtypeStruct((M, M), jnp.float32),
        jax.ShapeDtypeStruct((M, M), jnp.float32),
    ),
)


def _topk_body(x_ref, z_ref, idx_ref, kxz_ref):
    x = x_ref[...]
    z = z_ref[...]
    xn = jnp.sum(x * x, axis=1)[:, None]
    zn = jnp.sum(z * z, axis=1)[None, :]
    g = jnp.dot(x, z.T, preferred_element_type=jnp.float32)
    work = jnp.maximum(xn + zn - 2.0 * g, 0.0)
    col = lax.broadcasted_iota(jnp.int32, (RB, M), 1)
    big = jnp.int32(1 << 30)
    idx_cols = []
    val_cols = []
    for _ in range(K):
        m = jnp.min(work, axis=1, keepdims=True)
        cand = jnp.where(work == m, col, big)
        ai = jnp.min(cand, axis=1, keepdims=True)
        idx_cols.append(ai)
        val_cols.append(m)
        work = jnp.where(col == ai, jnp.inf, work)
    idx_blk = jnp.concatenate(idx_cols, axis=1)
    val_blk = jnp.concatenate(val_cols, axis=1)
    idx_ref[...] = idx_blk.T
    kxz_ref[...] = jnp.exp(-0.5 * val_blk).T


_topk = pl.pallas_call(
    _topk_body,
    grid=(N // RB,),
    in_specs=[
        pl.BlockSpec((RB, D), lambda i: (i, 0)),
        pl.BlockSpec((M, D), lambda i: (0, 0)),
    ],
    out_specs=(
        pl.BlockSpec((K, RB), lambda i: (0, i)),
        pl.BlockSpec((K, RB), lambda i: (0, i)),
    ),
    out_shape=(
        jax.ShapeDtypeStruct((K, N), jnp.int32),
        jax.ShapeDtypeStruct((K, N), jnp.float32),
    ),
)


def _sc_gather_body(idx_hbm, th_hbm, td_hbm, mu_hbm, out_hbm,
                    idxv, fidx, gbuf, sem):
    c = lax.axis_index("c")
    s = lax.axis_index("s")
    wid = s * 2 + c
    base = wid * RW
    pltpu.sync_copy(idx_hbm.at[:, pl.ds(base, RW)], idxv)

    for a in range(K):
        @pl.loop(0, RW // 16)
        def _(jc, a=a):
            off = jc * 16
            va = idxv[a, pl.ds(off, 16)] * M
            for b in range(K):
                fidx[a * K + b, pl.ds(off, 16)] = va + idxv[b, pl.ds(off, 16)]

    def _start_h(p):
        pltpu.async_copy(th_hbm.at[fidx.at[p]], gbuf.at[p], sem)

    def _wait_h(p):
        pltpu.make_async_copy(th_hbm.at[fidx.at[p]], gbuf.at[p], sem).wait()

    def _start_d(p):
        pltpu.async_copy(td_hbm.at[fidx.at[p]], gbuf.at[NP + p], sem)

    def _wait_d(p):
        pltpu.make_async_copy(td_hbm.at[fidx.at[p]], gbuf.at[NP + p],
                              sem).wait()

    for d in range(DEPTH):
        _start_h(d)

    @pl.loop(0, NP - DEPTH)
    def _(p):
        _wait_h(p)
        _start_h(p + DEPTH)

    for d in range(DEPTH):
        _wait_h(NP - DEPTH + d)
        _start_d(d)

    @pl.loop(0, NP - DEPTH)
    def _(p):
        _wait_d(p)
        _start_d(p + DEPTH)

    for d in range(DEPTH):
        _wait_d(NP - DEPTH + d)
    for a in range(K):
        pltpu.async_copy(mu_hbm.at[idxv.at[a]], gbuf.at[2 * NP + a], sem)
    for a in range(K):
        pltpu.make_async_copy(mu_hbm.at[idxv.at[a]], gbuf.at[2 * NP + a],
                              sem).wait()

    pltpu.sync_copy(gbuf, out_hbm.at[:, pl.ds(base, RW)])


@functools.lru_cache(maxsize=1)
def _sc_gather():
    return functools.partial(
        pl.kernel,
        out_type=jax.ShapeDtypeStruct((GROWS, N), jnp.float32),
        mesh=plsc.VectorSubcoreMesh(core_axis_name="c", subcore_axis_name="s"),
        scratch_types=[
            pltpu.VMEM((K, RW), jnp.int32),
            pltpu.VMEM((NP, RW), jnp.int32),
            pltpu.VMEM((GROWS, RW), jnp.float32),
            pltpu.SemaphoreType.DMA,
        ],
    )(_sc_gather_body)


def _solve_body(g_ref, b_ref, mean_ref, cov_ref, qf_ref):
    a = {}
    for i in range(K):
        for j in range(i + 1):
            a[(i, j)] = g_ref[i * K + j]
    l = {}
    rinv = [None] * K
    for k in range(K):
        r = lax.rsqrt(a[(k, k)])
        rinv[k] = r
        for i in range(k + 1, K):
            l[(i, k)] = a[(i, k)] * r
        for j in range(k + 1, K):
            for i in range(j, K):
                a[(i, j)] = a[(i, j)] - l[(i, k)] * l[(j, k)]
    y = []
    for i in range(K):
        acc = b_ref[i]
        for k2 in range(i):
            acc = acc - l[(i, k2)] * y[k2]
        y.append(acc * rinv[i])
    w = [None] * K
    for i in reversed(range(K)):
        acc = y[i]
        for k2 in range(i + 1, K):
            acc = acc - l[(k2, i)] * w[k2]
        w[i] = acc * rinv[i]
    q = None
    for i in range(K):
        ti = None
        for j in range(K):
            dij = g_ref[NP + i * K + j]
            ti = dij * w[j] if ti is None else ti + dij * w[j]
        q = ti * w[i] if q is None else q + ti * w[i]
    mean = None
    for i in range(K):
        mean = (g_ref[2 * NP + i] * w[i] if mean is None
                else mean + g_ref[2 * NP + i] * w[i])
    cov = 1.0 + q
    mean_ref[...] = mean
    cov_ref[...] = cov
    qf_ref[...] = jnp.sqrt(jnp.maximum(cov, 0.05))


_solve = pl.pallas_call(
    _solve_body,
    out_shape=(
        jax.ShapeDtypeStruct((SUB, 512), jnp.float32),
        jax.ShapeDtypeStruct((SUB, 512), jnp.float32),
        jax.ShapeDtypeStruct((SUB, 512), jnp.float32),
    ),
)


def kernel(X, Z, Lu_raw, mu):
    th, td, lu = _tables(Z, Lu_raw)
    idx_t, kxz_t = _topk(X, Z)
    g_all = _sc_gather()(idx_t, th.reshape(M * M), td.reshape(M * M), mu)
    mean8, cov8, qf8 = _solve(
        g_all.reshape(GROWS, SUB, 512), kxz_t.reshape(K, SUB, 512))
    mean = mean8.reshape(1, N)
    cov = cov8.reshape(1, N)
    qf = qf8.reshape(1, N)
    return (mean, qf, cov, mu, lu)


# speedup vs baseline: 1.2282x; 1.2282x over previous
import functools

import jax
import jax.numpy as jnp
from jax import lax
from jax.experimental import pallas as pl
from jax.experimental.pallas import tpu as pltpu
from jax.experimental.pallas import tpu_sc as plsc

N, D, M, K = 4096, 32, 512, 16
JITTER = 1e-4
RB = 256
NW = 32
RW = N // NW
NT = K * (K + 1) // 2
GROWS = 2 * NT + K
SUB = N // 512
DEPTH = 16


def _tables_body(z_ref, lu_raw_ref, th_ref, td_ref, lu_ref):
    z = z_ref[...]
    zn = jnp.sum(z * z, axis=1)
    g = jnp.dot(z, z.T, preferred_element_type=jnp.float32)
    d2 = jnp.maximum(zn[:, None] + zn[None, :] - 2.0 * g, 0.0)
    kzz = jnp.exp(-0.5 * d2)
    ri = lax.broadcasted_iota(jnp.int32, (M, M), 0)
    ci = lax.broadcasted_iota(jnp.int32, (M, M), 1)
    eye = (ri == ci).astype(jnp.float32)
    lur = lu_raw_ref[...]
    lu = jnp.where(ri > ci, lur, 0.0) + eye * jnp.exp(lur * eye)
    s = jnp.dot(lu, lu.T, preferred_element_type=jnp.float32)
    th_ref[...] = kzz + (2.0 * JITTER) * eye
    td_ref[...] = s - kzz - JITTER * eye
    lu_ref[...] = lu


_tables = pl.pallas_call(
    _tables_body,
    out_shape=(
        jax.ShapeDtypeStruct((M, M), jnp.float32),
        jax.ShapeDtypeStruct((M, M), jnp.float32),
        jax.ShapeDtypeStruct((M, M), jnp.float32),
    ),
)


def _topk_body(x_ref, z_ref, idx_ref, kxz_ref):
    x = x_ref[...]
    z = z_ref[...]
    xn = jnp.sum(x * x, axis=1)[:, None]
    zn = jnp.sum(z * z, axis=1)[None, :]
    g = jnp.dot(x, z.T, preferred_element_type=jnp.float32)
    work = jnp.maximum(xn + zn - 2.0 * g, 0.0)
    col = lax.broadcasted_iota(jnp.int32, (RB, M), 1)
    big = jnp.int32(1 << 30)
    idx_cols = []
    val_cols = []
    for _ in range(K):
        m = jnp.min(work, axis=1, keepdims=True)
        cand = jnp.where(work == m, col, big)
        ai = jnp.min(cand, axis=1, keepdims=True)
        idx_cols.append(ai)
        val_cols.append(m)
        work = jnp.where(col == ai, jnp.inf, work)
    idx_blk = jnp.concatenate(idx_cols, axis=1)
    val_blk = jnp.concatenate(val_cols, axis=1)
    idx_ref[...] = idx_blk.T
    kxz_ref[...] = jnp.exp(-0.5 * val_blk).T


_topk = pl.pallas_call(
    _topk_body,
    grid=(N // RB,),
    in_specs=[
        pl.BlockSpec((RB, D), lambda i: (i, 0)),
        pl.BlockSpec((M, D), lambda i: (0, 0)),
    ],
    out_specs=(
        pl.BlockSpec((K, RB), lambda i: (0, i)),
        pl.BlockSpec((K, RB), lambda i: (0, i)),
    ),
    out_shape=(
        jax.ShapeDtypeStruct((K, N), jnp.int32),
        jax.ShapeDtypeStruct((K, N), jnp.float32),
    ),
)


def _sc_gather_body(idx_hbm, th_hbm, td_hbm, mu_hbm, out_hbm,
                    idxv, fidx, gbuf, sem):
    c = lax.axis_index("c")
    s = lax.axis_index("s")
    wid = s * 2 + c
    base = wid * RW
    pltpu.sync_copy(idx_hbm.at[:, pl.ds(base, RW)], idxv)

    for i in range(K):
        @pl.loop(0, RW // 16)
        def _(jc, i=i):
            off = jc * 16
            vi = idxv[i, pl.ds(off, 16)] * M
            for j in range(i + 1):
                fidx[i * (i + 1) // 2 + j, pl.ds(off, 16)] = (
                    vi + idxv[j, pl.ds(off, 16)])

    def _start_h(p):
        pltpu.async_copy(th_hbm.at[fidx.at[p]], gbuf.at[p], sem)

    def _wait_h(p):
        pltpu.make_async_copy(th_hbm.at[fidx.at[p]], gbuf.at[p], sem).wait()

    def _start_d(p):
        pltpu.async_copy(td_hbm.at[fidx.at[p]], gbuf.at[NT + p], sem)

    def _wait_d(p):
        pltpu.make_async_copy(td_hbm.at[fidx.at[p]], gbuf.at[NT + p],
                              sem).wait()

    for d in range(DEPTH):
        _start_h(d)

    @pl.loop(0, NT - DEPTH)
    def _(p):
        _wait_h(p)
        _start_h(p + DEPTH)

    for d in range(DEPTH):
        _wait_h(NT - DEPTH + d)
        _start_d(d)

    @pl.loop(0, NT - DEPTH)
    def _(p):
        _wait_d(p)
        _start_d(p + DEPTH)

    for d in range(DEPTH):
        _wait_d(NT - DEPTH + d)
    for a in range(K):
        pltpu.async_copy(mu_hbm.at[idxv.at[a]], gbuf.at[2 * NT + a], sem)
    for a in range(K):
        pltpu.make_async_copy(mu_hbm.at[idxv.at[a]], gbuf.at[2 * NT + a],
                              sem).wait()

    pltpu.sync_copy(gbuf, out_hbm.at[:, pl.ds(base, RW)])


@functools.lru_cache(maxsize=1)
def _sc_gather():
    return functools.partial(
        pl.kernel,
        out_type=jax.ShapeDtypeStruct((GROWS, N), jnp.float32),
        mesh=plsc.VectorSubcoreMesh(core_axis_name="c", subcore_axis_name="s"),
        scratch_types=[
            pltpu.VMEM((K, RW), jnp.int32),
            pltpu.VMEM((NT, RW), jnp.int32),
            pltpu.VMEM((GROWS, RW), jnp.float32),
            pltpu.SemaphoreType.DMA,
        ],
    )(_sc_gather_body)


def _tri(i, j):
    return i * (i + 1) // 2 + j


def _solve_body(g_ref, b_ref, mean_ref, cov_ref, qf_ref):
    a = {}
    for i in range(K):
        for j in range(i + 1):
            a[(i, j)] = g_ref[_tri(i, j)]
    l = {}
    rinv = [None] * K
    for k in range(K):
        r = lax.rsqrt(a[(k, k)])
        rinv[k] = r
        for i in range(k + 1, K):
            l[(i, k)] = a[(i, k)] * r
        for j in range(k + 1, K):
            for i in range(j, K):
                a[(i, j)] = a[(i, j)] - l[(i, k)] * l[(j, k)]
    y = []
    for i in range(K):
        acc = b_ref[i]
        for k2 in range(i):
            acc = acc - l[(i, k2)] * y[k2]
        y.append(acc * rinv[i])
    w = [None] * K
    for i in reversed(range(K)):
        acc = y[i]
        for k2 in range(i + 1, K):
            acc = acc - l[(k2, i)] * w[k2]
        w[i] = acc * rinv[i]
    q = None
    for i in range(K):
        ti = None
        for j in range(i):
            dij = g_ref[NT + _tri(i, j)]
            ti = dij * w[j] if ti is None else ti + dij * w[j]
        dii = g_ref[NT + _tri(i, i)]
        ti = dii * w[i] if ti is None else 2.0 * ti + dii * w[i]
        q = ti * w[i] if q is None else q + ti * w[i]
    mean = None
    for i in range(K):
        mean = (g_ref[2 * NT + i] * w[i] if mean is None
                else mean + g_ref[2 * NT + i] * w[i])
    cov = 1.0 + q
    mean_ref[...] = mean
    cov_ref[...] = cov
    qf_ref[...] = jnp.sqrt(jnp.maximum(cov, 0.05))


_solve = pl.pallas_call(
    _solve_body,
    out_shape=(
        jax.ShapeDtypeStruct((SUB, 512), jnp.float32),
        jax.ShapeDtypeStruct((SUB, 512), jnp.float32),
        jax.ShapeDtypeStruct((SUB, 512), jnp.float32),
    ),
)


def kernel(X, Z, Lu_raw, mu):
    th, td, lu = _tables(Z, Lu_raw)
    idx_t, kxz_t = _topk(X, Z)
    g_all = _sc_gather()(idx_t, th.reshape(M * M), td.reshape(M * M), mu)
    mean8, cov8, qf8 = _solve(
        g_all.reshape(GROWS, SUB, 512), kxz_t.reshape(K, SUB, 512))
    mean = mean8.reshape(1, N)
    cov = cov8.reshape(1, N)
    qf = qf8.reshape(1, N)
    return (mean, qf, cov, mu, lu)


# speedup vs baseline: 1.3351x; 1.0870x over previous
import functools

import jax
import jax.numpy as jnp
from jax import lax
from jax.experimental import pallas as pl
from jax.experimental.pallas import tpu as pltpu
from jax.experimental.pallas import tpu_sc as plsc

N, D, M, K = 4096, 32, 512, 16
JITTER = 1e-4
RB = 256
NW = 32
RW = N // NW
NT = K * (K + 1) // 2
GROWS = 2 * NT + K
SUB = N // 512
DEPTH = 32


def _tables_body(z_ref, lu_raw_ref, th_ref, td_ref, lu_ref):
    z = z_ref[...]
    zn = jnp.sum(z * z, axis=1)
    g = jnp.dot(z, z.T, preferred_element_type=jnp.float32)
    d2 = jnp.maximum(zn[:, None] + zn[None, :] - 2.0 * g, 0.0)
    kzz = jnp.exp(-0.5 * d2)
    ri = lax.broadcasted_iota(jnp.int32, (M, M), 0)
    ci = lax.broadcasted_iota(jnp.int32, (M, M), 1)
    eye = (ri == ci).astype(jnp.float32)
    lur = lu_raw_ref[...]
    lu = jnp.where(ri > ci, lur, 0.0) + eye * jnp.exp(lur * eye)
    s = jnp.dot(lu, lu.T, preferred_element_type=jnp.float32)
    th_ref[...] = kzz + (2.0 * JITTER) * eye
    td_ref[...] = s - kzz - JITTER * eye
    lu_ref[...] = lu


_tables = pl.pallas_call(
    _tables_body,
    out_shape=(
        jax.ShapeDtypeStruct((M, M), jnp.float32),
        jax.ShapeDtypeStruct((M, M), jnp.float32),
        jax.ShapeDtypeStruct((M, M), jnp.float32),
    ),
)


def _topk_body(x_ref, z_ref, idx_ref, kxz_ref):
    x = x_ref[...]
    z = z_ref[...]
    xn = jnp.sum(x * x, axis=1)[:, None]
    zn = jnp.sum(z * z, axis=1)[None, :]
    g = jnp.dot(x, z.T, preferred_element_type=jnp.float32)
    d2 = jnp.maximum(xn + zn - 2.0 * g, 0.0)
    col = lax.broadcasted_iota(jnp.int32, (RB, M), 1)
    work = (lax.bitcast_convert_type(d2, jnp.int32) & jnp.int32(-512)) | col
    big = jnp.int32(0x7FFFFFFF)
    key_cols = []
    for _ in range(K):
        m = jnp.min(work, axis=1, keepdims=True)
        key_cols.append(m)
        work = jnp.where(work == m, big, work)
    keys = jnp.concatenate(key_cols, axis=1)
    idx_blk = keys & jnp.int32(511)
    val_blk = lax.bitcast_convert_type(keys & jnp.int32(-512), jnp.float32)
    idx_ref[...] = idx_blk.T
    kxz_ref[...] = jnp.exp(-0.5 * val_blk).T


_topk = pl.pallas_call(
    _topk_body,
    grid=(N // RB,),
    in_specs=[
        pl.BlockSpec((RB, D), lambda i: (i, 0)),
        pl.BlockSpec((M, D), lambda i: (0, 0)),
    ],
    out_specs=(
        pl.BlockSpec((K, RB), lambda i: (0, i)),
        pl.BlockSpec((K, RB), lambda i: (0, i)),
    ),
    out_shape=(
        jax.ShapeDtypeStruct((K, N), jnp.int32),
        jax.ShapeDtypeStruct((K, N), jnp.float32),
    ),
)


def _sc_gather_body(idx_hbm, th_hbm, td_hbm, mu_hbm, out_hbm,
                    idxv, fidx, gbuf, sem):
    c = lax.axis_index("c")
    s = lax.axis_index("s")
    wid = s * 2 + c
    base = wid * RW
    pltpu.sync_copy(idx_hbm.at[:, pl.ds(base, RW)], idxv)

    for i in range(K):
        @pl.loop(0, RW // 16)
        def _(jc, i=i):
            off = jc * 16
            vi = idxv[i, pl.ds(off, 16)] * M
            for j in range(i + 1):
                fidx[i * (i + 1) // 2 + j, pl.ds(off, 16)] = (
                    vi + idxv[j, pl.ds(off, 16)])

    def _start_h(p):
        pltpu.async_copy(th_hbm.at[fidx.at[p]], gbuf.at[p], sem)

    def _wait_h(p):
        pltpu.make_async_copy(th_hbm.at[fidx.at[p]], gbuf.at[p], sem).wait()

    def _start_d(p):
        pltpu.async_copy(td_hbm.at[fidx.at[p]], gbuf.at[NT + p], sem)

    def _wait_d(p):
        pltpu.make_async_copy(td_hbm.at[fidx.at[p]], gbuf.at[NT + p],
                              sem).wait()

    for d in range(DEPTH):
        _start_h(d)

    @pl.loop(0, NT - DEPTH)
    def _(p):
        _wait_h(p)
        _start_h(p + DEPTH)

    for d in range(DEPTH):
        _wait_h(NT - DEPTH + d)
        _start_d(d)

    @pl.loop(0, NT - DEPTH)
    def _(p):
        _wait_d(p)
        _start_d(p + DEPTH)

    for d in range(DEPTH):
        _wait_d(NT - DEPTH + d)
    for a in range(K):
        pltpu.async_copy(mu_hbm.at[idxv.at[a]], gbuf.at[2 * NT + a], sem)
    for a in range(K):
        pltpu.make_async_copy(mu_hbm.at[idxv.at[a]], gbuf.at[2 * NT + a],
                              sem).wait()

    pltpu.sync_copy(gbuf, out_hbm.at[:, pl.ds(base, RW)])


@functools.lru_cache(maxsize=1)
def _sc_gather():
    return functools.partial(
        pl.kernel,
        out_type=jax.ShapeDtypeStruct((GROWS, N), jnp.float32),
        mesh=plsc.VectorSubcoreMesh(core_axis_name="c", subcore_axis_name="s"),
        scratch_types=[
            pltpu.VMEM((K, RW), jnp.int32),
            pltpu.VMEM((NT, RW), jnp.int32),
            pltpu.VMEM((GROWS, RW), jnp.float32),
            pltpu.SemaphoreType.DMA,
        ],
    )(_sc_gather_body)


def _tri(i, j):
    return i * (i + 1) // 2 + j


def _solve_body(g_ref, b_ref, mean_ref, cov_ref, qf_ref):
    a = {}
    for i in range(K):
        for j in range(i + 1):
            a[(i, j)] = g_ref[_tri(i, j)]
    l = {}
    rinv = [None] * K
    for k in range(K):
        r = lax.rsqrt(a[(k, k)])
        rinv[k] = r
        for i in range(k + 1, K):
            l[(i, k)] = a[(i, k)] * r
        for j in range(k + 1, K):
            for i in range(j, K):
                a[(i, j)] = a[(i, j)] - l[(i, k)] * l[(j, k)]
    y = []
    for i in range(K):
        acc = b_ref[i]
        for k2 in range(i):
            acc = acc - l[(i, k2)] * y[k2]
        y.append(acc * rinv[i])
    w = [None] * K
    for i in reversed(range(K)):
        acc = y[i]
        for k2 in range(i + 1, K):
            acc = acc - l[(k2, i)] * w[k2]
        w[i] = acc * rinv[i]
    q = None
    for i in range(K):
        ti = None
        for j in range(i):
            dij = g_ref[NT + _tri(i, j)]
            ti = dij * w[j] if ti is None else ti + dij * w[j]
        dii = g_ref[NT + _tri(i, i)]
        ti = dii * w[i] if ti is None else 2.0 * ti + dii * w[i]
        q = ti * w[i] if q is None else q + ti * w[i]
    mean = None
    for i in range(K):
        mean = (g_ref[2 * NT + i] * w[i] if mean is None
                else mean + g_ref[2 * NT + i] * w[i])
    cov = 1.0 + q
    mean_ref[...] = mean
    cov_ref[...] = cov
    qf_ref[...] = jnp.sqrt(jnp.maximum(cov, 0.05))


_solve = pl.pallas_call(
    _solve_body,
    out_shape=(
        jax.ShapeDtypeStruct((SUB, 512), jnp.float32),
        jax.ShapeDtypeStruct((SUB, 512), jnp.float32),
        jax.ShapeDtypeStruct((SUB, 512), jnp.float32),
    ),
)


def kernel(X, Z, Lu_raw, mu):
    th, td, lu = _tables(Z, Lu_raw)
    idx_t, kxz_t = _topk(X, Z)
    g_all = _sc_gather()(idx_t, th.reshape(M * M), td.reshape(M * M), mu)
    mean8, cov8, qf8 = _solve(
        g_all.reshape(GROWS, SUB, 512), kxz_t.reshape(K, SUB, 512))
    mean = mean8.reshape(1, N)
    cov = cov8.reshape(1, N)
    qf = qf8.reshape(1, N)
    return (mean, qf, cov, mu, lu)
